# baseline (speedup 1.0000x reference)
import functools

import jax
import jax.numpy as jnp
import numpy as np
from jax import lax
from jax.experimental import pallas as pl
from jax.experimental.pallas import tpu as pltpu
from jax.experimental.pallas import tpu_sc as plsc

_NC = 2
_NS = 16
_NW = _NC * _NS
_LANES = 16

_BINS = 10
_SLOTS = (_BINS + 1) * _LANES
_HIST = 2 * _SLOTS

_P0 = 9.9930139e-01
_P1 = -4.8463577e-01
_P2 = 2.5187504e-01
_P3 = -7.3899068e-02
_LOG2E = 1.4426950408889634


def _sc_hist_kernel(n_total: int, chunk: int, unroll: int):
    per_w = n_total // _NW
    nchunk = per_w // chunk
    nvec = chunk // _LANES
    assert nchunk % 2 == 0 and nchunk >= 4

    mesh = plsc.VectorSubcoreMesh(core_axis_name="c", subcore_axis_name="s")

    @functools.partial(
        pl.kernel,
        mesh=mesh,
        out_type=jax.ShapeDtypeStruct((_NW, _HIST), jnp.float32),
        compiler_params=pltpu.CompilerParams(needs_layout_passes=False),
        scratch_types=[
            pltpu.VMEM((chunk,), jnp.float32),
            pltpu.VMEM((chunk,), jnp.float32),
            pltpu.VMEM((chunk,), jnp.int32),
            pltpu.VMEM((chunk,), jnp.int32),
            pltpu.VMEM((_HIST,), jnp.float32),
            pltpu.SemaphoreType.DMA((2,)),
        ],
    )
    def k(pred_hbm, tgt_hbm, out_hbm, xbuf0, xbuf1, tbuf0, tbuf1, hist, sem):
        xbufs = (xbuf0, xbuf1)
        tbufs = (tbuf0, tbuf1)
        wid = lax.axis_index("s") * _NC + lax.axis_index("c")
        base = wid * per_w

        zeros = jnp.zeros((_LANES,), jnp.float32)
        for i in range(_HIST // _LANES):
            hist[pl.ds(i * _LANES, _LANES)] = zeros

        lane = lax.iota(jnp.int32, _LANES)
        ones = jnp.ones((_LANES,), jnp.float32)

        def fire(slot, c):
            off = base + c * chunk
            pltpu.async_copy(pred_hbm.at[pl.ds(off, chunk)],
                             xbufs[slot], sem.at[slot])
            pltpu.async_copy(tgt_hbm.at[pl.ds(off, chunk)],
                             tbufs[slot], sem.at[slot])

        def drain(slot):
            pltpu.make_async_copy(pred_hbm.at[pl.ds(base, chunk)],
                                  xbufs[slot], sem.at[slot]).wait()
            pltpu.make_async_copy(tgt_hbm.at[pl.ds(base, chunk)],
                                  tbufs[slot], sem.at[slot]).wait()

        def compute(slot):
            xr = xbufs[slot]
            tr = tbufs[slot]

            @plsc.parallel_loop(0, nvec, unroll=unroll)
            def _vec(i):
                x = xr[pl.ds(i * _LANES, _LANES)]
                t = tr[pl.ds(i * _LANES, _LANES)]
                ax = jnp.abs(x)
                a = jnp.exp(-ax)
                mv = (t == 1) != (x > 0.0)
                inv = 1.0 / (1.0 + a)
                g = jnp.where(mv, inv, a * inv)
                l1p = _P3 * a + _P2
                l1p = l1p * a + _P1
                l1p = l1p * a + _P0
                l1p = l1p * a
                le = jnp.where(mv, ax, 0.0) + l1p
                k_ = (g * 10.0).astype(jnp.int32)
                fidx = k_ * _LANES + lane
                plsc.addupdate_scatter(hist, [fidx], ones)
                plsc.addupdate_scatter(hist, [fidx + _SLOTS], le)

        fire(0, 0)
        fire(1, 1)

        @pl.loop(0, nchunk, step=2)
        def _chunks(c):
            for s in range(2):
                drain(s)
                compute(s)

                @pl.when(c + s + 2 < nchunk)
                def _():
                    fire(s, c + s + 2)

        pltpu.sync_copy(hist, out_hbm.at[wid])

    return k


def _tc_hist_kernel(n_start: int, n_stop: int, w: int):
    bl = 8 * w
    steps = (n_stop - n_start) // bl
    assert steps % 2 == 0 and steps >= 4 and (n_stop - n_start) % bl == 0
    cols = w // 128

    def body(x_hbm, t_hbm, out_ref, xb0, xb1, tb0, tb1, sem):
        xbufs = (xb0, xb1)
        tbufs = (tb0, tb1)

        def fire(slot, step):
            off = n_start + step * bl
            for r in range(8):
                pltpu.make_async_copy(
                    x_hbm.at[pl.ds(off + r * w, w)],
                    xbufs[slot].at[r], sem.at[slot]).start()
                pltpu.make_async_copy(
                    t_hbm.at[pl.ds(off + r * w, w)],
                    tbufs[slot].at[r], sem.at[slot]).start()

        def drain(slot):
            for r in range(8):
                pltpu.make_async_copy(
                    x_hbm.at[pl.ds(n_start, w)],
                    xbufs[slot].at[r], sem.at[slot]).wait()
                pltpu.make_async_copy(
                    t_hbm.at[pl.ds(n_start, w)],
                    tbufs[slot].at[r], sem.at[slot]).wait()

        fire(0, 0)
        fire(1, 1)

        def step_fn(i, acc):
            for s in range(2):
                drain(s)
                xr, tr = xbufs[s], tbufs[s]

                def col_fn(j, acc):
                    x = xr[:, pl.ds(j * 128, 128)]
                    t = tr[:, pl.ds(j * 128, 128)]
                    ax = jnp.abs(x)
                    a = jnp.exp(-ax)
                    mv = (t == 1) != (x > 0.0)
                    inv = 1.0 / (1.0 + a)
                    g = jnp.where(mv, inv, a * inv)
                    le = jnp.where(mv, ax, 0.0) + jnp.log1p(a)
                    k_ = (g * 10.0).astype(jnp.int32)
                    cnts, lss = acc
                    new_c = []
                    new_l = []
                    for b in range(_BINS):
                        m = k_ == b
                        new_c.append(cnts[b] + m.astype(jnp.float32))
                        new_l.append(lss[b] + jnp.where(m, le, 0.0))
                    return tuple(new_c), tuple(new_l)

                acc = lax.fori_loop(0, cols, col_fn, acc)

                @pl.when(2 * i + s + 2 < steps)
                def _():
                    fire(s, 2 * i + s + 2)
            return acc

        zero = jnp.zeros((8, 128), jnp.float32)
        acc0 = (tuple(zero for _ in range(_BINS)),
                tuple(zero for _ in range(_BINS)))
        cnts, lss = lax.fori_loop(0, steps // 2, step_fn, acc0)

        for b in range(_BINS):
            out_ref[b, :] = jnp.sum(cnts[b], axis=0)
            out_ref[_BINS + 1 + b, :] = jnp.sum(lss[b], axis=0)
        out_ref[_BINS, :] = jnp.zeros((128,), jnp.float32)
        out_ref[2 * _BINS + 1, :] = jnp.zeros((128,), jnp.float32)

    return pl.pallas_call(
        body,
        in_specs=[pl.BlockSpec(memory_space=pl.MemorySpace.ANY),
                  pl.BlockSpec(memory_space=pl.MemorySpace.ANY)],
        out_shape=jax.ShapeDtypeStruct((2 * _BINS + 2, 128), jnp.float32),
        scratch_shapes=[
            pltpu.VMEM((8, w), jnp.float32),
            pltpu.VMEM((8, w), jnp.float32),
            pltpu.VMEM((8, w), jnp.int32),
            pltpu.VMEM((8, w), jnp.int32),
            pltpu.SemaphoreType.DMA((2,)),
        ],
    )


def _combine_kernel(part_ref, sel_ref, tc_ref, out_ref):
    tot = jnp.sum(part_ref[...], axis=0, keepdims=True)
    s = jnp.dot(tot, sel_ref[...],
                preferred_element_type=jnp.float32)
    s = s[0, :] + jnp.sum(tc_ref[...], axis=1)
    cnt = s[0:_BINS]
    ls = s[_BINS + 1:2 * _BINS + 1]
    nz = cnt > 0.0
    n = jnp.sum(nz.astype(jnp.float32))
    terms = jnp.where(nz, ls / jnp.maximum(cnt, 1.0), 0.0)
    loss = jnp.where(n > 0.0, jnp.sum(terms) / jnp.maximum(n, 1.0), 0.0)
    out_ref[...] = jnp.reshape(loss, (1, 1))


_SC_SHARE = 8 * 2 ** 20


def kernel(pred, target):
    n_total = pred.shape[0]
    t = jnp.reshape(target, (-1,))
    parts = _sc_hist_kernel(_SC_SHARE, 16384, 8)(pred, t)
    tc_parts = _tc_hist_kernel(_SC_SHARE, n_total, 8192)(pred, t)
    ngroups = _HIST // _LANES
    sel = jnp.asarray(
        (np.arange(_HIST)[:, None] // _LANES
         == np.arange(ngroups)[None, :]).astype(np.float32))
    out = pl.pallas_call(
        _combine_kernel,
        out_shape=jax.ShapeDtypeStruct((1, 1), jnp.float32),
    )(parts, sel, tc_parts)
    return out[0, 0]


# speedup vs baseline: 1.5311x; 1.5311x over previous
import functools

import jax
import jax.numpy as jnp
import numpy as np
from jax import lax
from jax.experimental import pallas as pl
from jax.experimental.pallas import tpu as pltpu
from jax.experimental.pallas import tpu_sc as plsc

_NC = 2
_NS = 16
_NW = _NC * _NS
_LANES = 16

_BINS = 10
_SLOTS = (_BINS + 1) * _LANES
_HIST = 2 * _SLOTS

_P0 = 9.9930139e-01
_P1 = -4.8463577e-01
_P2 = 2.5187504e-01
_P3 = -7.3899068e-02
_LOG2E = 1.4426950408889634


def _sc_hist_kernel(n_total: int, chunk: int, unroll: int):
    per_w = n_total // _NW
    nchunk = per_w // chunk
    nvec = chunk // _LANES
    assert nchunk % 2 == 0 and nchunk >= 4

    mesh = plsc.VectorSubcoreMesh(core_axis_name="c", subcore_axis_name="s")

    @functools.partial(
        pl.kernel,
        mesh=mesh,
        out_type=jax.ShapeDtypeStruct((_NW, _HIST), jnp.float32),
        compiler_params=pltpu.CompilerParams(needs_layout_passes=False),
        scratch_types=[
            pltpu.VMEM((chunk,), jnp.float32),
            pltpu.VMEM((chunk,), jnp.float32),
            pltpu.VMEM((chunk,), jnp.int32),
            pltpu.VMEM((chunk,), jnp.int32),
            pltpu.VMEM((_HIST,), jnp.float32),
            pltpu.SemaphoreType.DMA((2,)),
        ],
    )
    def k(pred_hbm, tgt_hbm, out_hbm, xbuf0, xbuf1, tbuf0, tbuf1, hist, sem):
        xbufs = (xbuf0, xbuf1)
        tbufs = (tbuf0, tbuf1)
        wid = lax.axis_index("s") * _NC + lax.axis_index("c")
        base = wid * per_w

        zeros = jnp.zeros((_LANES,), jnp.float32)
        for i in range(_HIST // _LANES):
            hist[pl.ds(i * _LANES, _LANES)] = zeros

        lane = lax.iota(jnp.int32, _LANES)
        ones = jnp.ones((_LANES,), jnp.float32)

        def fire(slot, c):
            off = base + c * chunk
            pltpu.async_copy(pred_hbm.at[pl.ds(off, chunk)],
                             xbufs[slot], sem.at[slot])
            pltpu.async_copy(tgt_hbm.at[pl.ds(off, chunk)],
                             tbufs[slot], sem.at[slot])

        def drain(slot):
            pltpu.make_async_copy(pred_hbm.at[pl.ds(base, chunk)],
                                  xbufs[slot], sem.at[slot]).wait()
            pltpu.make_async_copy(tgt_hbm.at[pl.ds(base, chunk)],
                                  tbufs[slot], sem.at[slot]).wait()

        def compute(slot):
            xr = xbufs[slot]
            tr = tbufs[slot]

            @plsc.parallel_loop(0, nvec, unroll=unroll)
            def _vec(i):
                x = xr[pl.ds(i * _LANES, _LANES)]
                t = tr[pl.ds(i * _LANES, _LANES)]
                ax = jnp.abs(x)
                a = jnp.exp(-ax)
                mv = (t == 1) != (x > 0.0)
                inv = 1.0 / (1.0 + a)
                g = jnp.where(mv, inv, a * inv)
                l1p = _P3 * a + _P2
                l1p = l1p * a + _P1
                l1p = l1p * a + _P0
                l1p = l1p * a
                le = jnp.where(mv, ax, 0.0) + l1p
                k_ = (g * 10.0).astype(jnp.int32)
                fidx = k_ * _LANES + lane
                plsc.addupdate_scatter(hist, [fidx], ones)
                plsc.addupdate_scatter(hist, [fidx + _SLOTS], le)

        fire(0, 0)
        fire(1, 1)

        @pl.loop(0, nchunk, step=2)
        def _chunks(c):
            for s in range(2):
                drain(s)
                compute(s)

                @pl.when(c + s + 2 < nchunk)
                def _():
                    fire(s, c + s + 2)

        pltpu.sync_copy(hist, out_hbm.at[wid])

    return k


def _combine_kernel(part_ref, sel_ref, out_ref):
    tot = jnp.sum(part_ref[...], axis=0, keepdims=True)
    s = jnp.dot(tot, sel_ref[...],
                preferred_element_type=jnp.float32)
    cnt = s[0, 0:_BINS]
    ls = s[0, _BINS + 1:2 * _BINS + 1]
    nz = cnt > 0.0
    n = jnp.sum(nz.astype(jnp.float32))
    terms = jnp.where(nz, ls / jnp.maximum(cnt, 1.0), 0.0)
    loss = jnp.where(n > 0.0, jnp.sum(terms) / jnp.maximum(n, 1.0), 0.0)
    out_ref[...] = jnp.reshape(loss, (1, 1))


def kernel(pred, target):
    n_total = pred.shape[0]
    t = jnp.reshape(target, (-1,))
    parts = _sc_hist_kernel(n_total, 16384, 6)(pred, t)
    ngroups = _HIST // _LANES
    sel = jnp.asarray(
        (np.arange(_HIST)[:, None] // _LANES
         == np.arange(ngroups)[None, :]).astype(np.float32))
    out = pl.pallas_call(
        _combine_kernel,
        out_shape=jax.ShapeDtypeStruct((1, 1), jnp.float32),
    )(parts, sel)
    return out[0, 0]


# speedup vs baseline: 1.6723x; 1.0922x over previous
import functools

import jax
import jax.numpy as jnp
import numpy as np
from jax import lax
from jax.experimental import pallas as pl
from jax.experimental.pallas import tpu as pltpu
from jax.experimental.pallas import tpu_sc as plsc

_NC = 2
_NS = 16
_NW = _NC * _NS
_LANES = 16

_BINS = 10
_SLOTS = (_BINS + 1) * _LANES
_HIST = 2 * _SLOTS

_P0 = 9.9560666e-01
_P1 = -4.4029690e-01
_P2 = 1.4102645e-01
_LOG2E = 1.4426950408889634


def _sc_hist_kernel(n_total: int, chunk: int, unroll: int):
    per_w = n_total // _NW
    nchunk = per_w // chunk
    nvec = chunk // _LANES
    assert nchunk % 2 == 0 and nchunk >= 4

    mesh = plsc.VectorSubcoreMesh(core_axis_name="c", subcore_axis_name="s")

    @functools.partial(
        pl.kernel,
        mesh=mesh,
        out_type=jax.ShapeDtypeStruct((_NW, _HIST), jnp.float32),
        compiler_params=pltpu.CompilerParams(needs_layout_passes=False),
        scratch_types=[
            pltpu.VMEM((chunk,), jnp.float32),
            pltpu.VMEM((chunk,), jnp.float32),
            pltpu.VMEM((chunk,), jnp.int32),
            pltpu.VMEM((chunk,), jnp.int32),
            pltpu.VMEM((_HIST,), jnp.float32),
            pltpu.SemaphoreType.DMA((2,)),
        ],
    )
    def k(pred_hbm, tgt_hbm, out_hbm, xbuf0, xbuf1, tbuf0, tbuf1, hist, sem):
        xbufs = (xbuf0, xbuf1)
        tbufs = (tbuf0, tbuf1)
        wid = lax.axis_index("s") * _NC + lax.axis_index("c")
        base = wid * per_w

        zeros = jnp.zeros((_LANES,), jnp.float32)
        for i in range(_HIST // _LANES):
            hist[pl.ds(i * _LANES, _LANES)] = zeros

        lane = lax.iota(jnp.int32, _LANES)
        ones = jnp.ones((_LANES,), jnp.float32)

        def fire(slot, c):
            off = base + c * chunk
            pltpu.async_copy(pred_hbm.at[pl.ds(off, chunk)],
                             xbufs[slot], sem.at[slot])
            pltpu.async_copy(tgt_hbm.at[pl.ds(off, chunk)],
                             tbufs[slot], sem.at[slot])

        def drain(slot):
            pltpu.make_async_copy(pred_hbm.at[pl.ds(base, chunk)],
                                  xbufs[slot], sem.at[slot]).wait()
            pltpu.make_async_copy(tgt_hbm.at[pl.ds(base, chunk)],
                                  tbufs[slot], sem.at[slot]).wait()

        def compute(slot):
            xr = xbufs[slot]
            tr = tbufs[slot]

            @plsc.parallel_loop(0, nvec, unroll=unroll)
            def _vec(i):
                x = xr[pl.ds(i * _LANES, _LANES)]
                t = tr[pl.ds(i * _LANES, _LANES)]
                ax = jnp.abs(x)
                a = jnp.exp(-ax)
                mv = (t == 1) != (x > 0.0)
                inv = 1.0 / (1.0 + a)
                g = jnp.where(mv, inv, a * inv)
                l1p = _P2 * a + _P1
                l1p = l1p * a + _P0
                l1p = l1p * a
                le = jnp.where(mv, ax, 0.0) + l1p
                k_ = (g * 10.0).astype(jnp.int32)
                fidx = k_ * _LANES + lane
                plsc.addupdate_scatter(hist, [fidx], ones)
                plsc.addupdate_scatter(hist, [fidx + _SLOTS], le)

        fire(0, 0)
        fire(1, 1)

        @pl.loop(0, nchunk, step=2)
        def _chunks(c):
            for s in range(2):
                drain(s)
                compute(s)

                @pl.when(c + s + 2 < nchunk)
                def _():
                    fire(s, c + s + 2)

        pltpu.sync_copy(hist, out_hbm.at[wid])

    return k


def _combine_kernel(part_ref, sel_ref, out_ref):
    tot = jnp.sum(part_ref[...], axis=0, keepdims=True)
    s = jnp.dot(tot, sel_ref[...],
                preferred_element_type=jnp.float32)
    cnt = s[0, 0:_BINS]
    ls = s[0, _BINS + 1:2 * _BINS + 1]
    nz = cnt > 0.0
    n = jnp.sum(nz.astype(jnp.float32))
    terms = jnp.where(nz, ls / jnp.maximum(cnt, 1.0), 0.0)
    loss = jnp.where(n > 0.0, jnp.sum(terms) / jnp.maximum(n, 1.0), 0.0)
    out_ref[...] = jnp.reshape(loss, (1, 1))


def kernel(pred, target):
    n_total = pred.shape[0]
    t = jnp.reshape(target, (-1,))
    parts = _sc_hist_kernel(n_total, 16384, 8)(pred, t)
    ngroups = _HIST // _LANES
    sel = jnp.asarray(
        (np.arange(_HIST)[:, None] // _LANES
         == np.arange(ngroups)[None, :]).astype(np.float32))
    out = pl.pallas_call(
        _combine_kernel,
        out_shape=jax.ShapeDtypeStruct((1, 1), jnp.float32),
    )(parts, sel)
    return out[0, 0]


# speedup vs baseline: 1.6726x; 1.0002x over previous
import functools

import jax
import jax.numpy as jnp
import numpy as np
from jax import lax
from jax.experimental import pallas as pl
from jax.experimental.pallas import tpu as pltpu
from jax.experimental.pallas import tpu_sc as plsc

_NC = 2
_NS = 16
_NW = _NC * _NS
_LANES = 16

_BINS = 10
_SLOTS = (_BINS + 1) * _LANES
_HIST = 2 * _SLOTS

_P0 = 9.9560666e-01
_P1 = -4.4029690e-01
_P2 = 1.4102645e-01


def _sc_hist_kernel(n_total: int, chunk: int, unroll: int):
    per_w = n_total // _NW
    nchunk = per_w // chunk
    nvec = chunk // _LANES
    assert nchunk % 2 == 0 and nchunk >= 4

    mesh = plsc.VectorSubcoreMesh(core_axis_name="c", subcore_axis_name="s")

    @functools.partial(
        pl.kernel,
        mesh=mesh,
        out_type=jax.ShapeDtypeStruct((_NW, _HIST), jnp.float32),
        compiler_params=pltpu.CompilerParams(needs_layout_passes=False),
        scratch_types=[
            pltpu.VMEM((chunk,), jnp.float32),
            pltpu.VMEM((chunk,), jnp.float32),
            pltpu.VMEM((chunk,), jnp.int32),
            pltpu.VMEM((chunk,), jnp.int32),
            pltpu.VMEM((_HIST,), jnp.float32),
            pltpu.SemaphoreType.DMA((2,)),
        ],
    )
    def k(pred_hbm, tgt_hbm, out_hbm, xbuf0, xbuf1, tbuf0, tbuf1, hist, sem):
        xbufs = (xbuf0, xbuf1)
        tbufs = (tbuf0, tbuf1)
        wid = lax.axis_index("s") * _NC + lax.axis_index("c")
        base = wid * per_w

        zeros = jnp.zeros((_LANES,), jnp.float32)
        for i in range(_HIST // _LANES):
            hist[pl.ds(i * _LANES, _LANES)] = zeros

        lane = lax.iota(jnp.int32, _LANES)
        ones = jnp.ones((_LANES,), jnp.float32)

        def fire(slot, c):
            off = base + c * chunk
            pltpu.async_copy(pred_hbm.at[pl.ds(off, chunk)],
                             xbufs[slot], sem.at[slot])
            pltpu.async_copy(tgt_hbm.at[pl.ds(off, chunk)],
                             tbufs[slot], sem.at[slot])

        def drain(slot):
            pltpu.make_async_copy(pred_hbm.at[pl.ds(base, chunk)],
                                  xbufs[slot], sem.at[slot]).wait()
            pltpu.make_async_copy(tgt_hbm.at[pl.ds(base, chunk)],
                                  tbufs[slot], sem.at[slot]).wait()

        def compute(slot):
            xr = xbufs[slot]
            tr = tbufs[slot]

            @plsc.parallel_loop(0, nvec, unroll=unroll)
            def _vec(i):
                x = xr[pl.ds(i * _LANES, _LANES)]
                t = tr[pl.ds(i * _LANES, _LANES)]
                ax = jnp.abs(x)
                a = jnp.exp(-ax)
                mv = (t == 1) != (x > 0.0)
                inv = 1.0 / (1.0 + a)
                g = jnp.where(mv, inv, a * inv)
                l1p = _P2 * a + _P1
                l1p = l1p * a + _P0
                l1p = l1p * a
                le = jnp.where(mv, ax, 0.0) + l1p
                k_ = (g * 10.0).astype(jnp.int32)
                fidx = k_ * _LANES + lane
                plsc.addupdate_scatter(hist, [fidx], ones)
                plsc.addupdate_scatter(hist, [fidx + _SLOTS], le)

        fire(0, 0)
        fire(1, 1)

        @pl.loop(0, nchunk, step=2)
        def _chunks(c):
            for s in range(2):
                drain(s)
                compute(s)

                @pl.when(c + s + 2 < nchunk)
                def _():
                    fire(s, c + s + 2)

        pltpu.sync_copy(hist, out_hbm.at[wid])

    return k


def _combine_kernel(part_ref, sel_ref, out_ref):
    tot = jnp.sum(part_ref[...], axis=0, keepdims=True)
    s = jnp.dot(tot, sel_ref[...],
                preferred_element_type=jnp.float32)
    cnt = s[0, 0:_BINS]
    ls = s[0, _BINS + 1:2 * _BINS + 1]
    nz = cnt > 0.0
    n = jnp.sum(nz.astype(jnp.float32))
    terms = jnp.where(nz, ls / jnp.maximum(cnt, 1.0), 0.0)
    loss = jnp.where(n > 0.0, jnp.sum(terms) / jnp.maximum(n, 1.0), 0.0)
    out_ref[...] = jnp.reshape(loss, (1, 1))


def kernel(pred, target):
    n_total = pred.shape[0]
    t = jnp.reshape(target, (-1,))
    parts = _sc_hist_kernel(n_total, 16384, 8)(pred, t)
    ngroups = _HIST // _LANES
    sel = jnp.asarray(
        (np.arange(_HIST)[:, None] // _LANES
         == np.arange(ngroups)[None, :]).astype(np.float32))
    out = pl.pallas_call(
        _combine_kernel,
        out_shape=jax.ShapeDtypeStruct((1, 1), jnp.float32),
    )(parts, sel)
    return out[0, 0]
